# baseline (speedup 1.0000x reference)
import functools

import jax
import jax.numpy as jnp
from jax import lax
from jax.experimental import pallas as pl
from jax.experimental.pallas import tpu as pltpu
from jax.experimental.pallas import tpu_sc as plsc

MIN_POS = -1.2
MAX_POS = 0.6
MAX_SPEED = 0.07

N = 16384
NC = 2
NS = 16
LANES = 16
IDX_BLK = 128

CH1 = 576
CH0 = 448
C1_TOTAL = NS * CH1
STG1 = CH1 + 64
STG0 = CH0 + 64

_B0 = float(-MIN_POS)
_B1 = float(MAX_SPEED)
_M0 = float(1023.999 / (MAX_POS - MIN_POS))
_M1 = float(1023.999 / (2.0 * MAX_SPEED))

_mesh = plsc.VectorSubcoreMesh(core_axis_name="c", subcore_axis_name="s")


def _blocks(chunk):
    out, off = [], 0
    while off < chunk:
        w = min(IDX_BLK, chunk - off)
        out.append((off, w))
        off += w
    return out


@functools.partial(
    pl.kernel,
    mesh=_mesh,
    out_type=jax.ShapeDtypeStruct((N,), jnp.float32),
    scratch_types=[
        pltpu.VMEM((STG1,), jnp.float32),
        pltpu.VMEM((STG1,), jnp.float32),
        pltpu.VMEM((CH1,), jnp.int32),
        pltpu.VMEM((CH1,), jnp.float32),
        pltpu.SemaphoreType.DMA,
        pltpu.SemaphoreType.DMA,
    ],
)
def _sc_lookup(inp_t_hbm, data_hbm, out_hbm, pos_v, vel_v, idx_v, out_v,
               sem_l, sem_g):
    core = lax.axis_index("c")
    sub = lax.axis_index("s")

    b0 = jnp.float32(_B0)
    b1 = jnp.float32(_B1)
    m0 = jnp.float32(_M0)
    m1 = jnp.float32(_M1)

    flat = data_hbm.at[0, pl.ds(0, IDX_BLK)]

    def _run(base, chunk, stage):
        intra = base % IDX_BLK
        abase = pl.multiple_of(base - intra, IDX_BLK)
        cp_p = pltpu.async_copy(
            inp_t_hbm.at[0, pl.ds(abase, stage)], pos_v.at[pl.ds(0, stage)],
            sem_l)
        cp_v = pltpu.async_copy(
            inp_t_hbm.at[1, pl.ds(abase, stage)], vel_v.at[pl.ds(0, stage)],
            sem_l)
        cp_p.wait()
        cp_v.wait()
        g_cps = []
        for off, w in _blocks(chunk):
            for g0 in range(off, off + w, LANES):
                pos = pos_v[pl.ds(g0 + intra, LANES)]
                vel = vel_v[pl.ds(g0 + intra, LANES)]
                r = ((pos + b0) * m0).astype(jnp.int32)
                c = ((vel + b1) * m1).astype(jnp.int32)
                idx_v[pl.ds(g0, LANES)] = (
                    ((r >> 3) << 13) + ((r & 7) << 7)
                    + ((c >> 7) << 10) + (c & 127)
                )
            g_cps.append(
                pltpu.async_copy(
                    flat.at[idx_v.at[pl.ds(off, w)]],
                    out_v.at[pl.ds(off, w)],
                    sem_g,
                )
            )
        for cp in g_cps:
            cp.wait()
        pltpu.sync_copy(
            out_v.at[pl.ds(0, chunk)], out_hbm.at[pl.ds(base, chunk)]
        )

    @pl.when(core == 1)
    def _():
        _run(sub * CH1, CH1, STG1)

    @pl.when(core == 0)
    def _():
        _run(C1_TOTAL + sub * CH0, CH0, STG0)


def kernel(inp, data):
    return _sc_lookup(inp.T, data)


# speedup vs baseline: 1.0758x; 1.0758x over previous
import functools

import jax
import jax.numpy as jnp
from jax import lax
from jax.experimental import pallas as pl
from jax.experimental.pallas import tpu as pltpu
from jax.experimental.pallas import tpu_sc as plsc

MIN_POS = -1.2
MAX_POS = 0.6
MAX_SPEED = 0.07

N = 16384
NC = 2
NS = 16
NW = NC * NS
CHUNK = N // NW
LANES = 16
IDX_BLK = 128
NBLK = CHUNK // IDX_BLK
GRP_PER_BLK = IDX_BLK // LANES

_B0 = float(-MIN_POS)
_B1 = float(MAX_SPEED)
_M0 = float(1023.999 / (MAX_POS - MIN_POS))
_M1 = float(1023.999 / (2.0 * MAX_SPEED))

_mesh = plsc.VectorSubcoreMesh(core_axis_name="c", subcore_axis_name="s")


@functools.partial(
    pl.kernel,
    mesh=_mesh,
    out_type=jax.ShapeDtypeStruct((N,), jnp.float32),
    scratch_types=[
        pltpu.VMEM((4, CHUNK), jnp.float32),
        pltpu.SemaphoreType.DMA,
        pltpu.SemaphoreType.DMA,
    ],
)
def _sc_lookup(inp_t_hbm, data_hbm, out_hbm, scr, sem_l, sem_g):
    wid = lax.axis_index("s") * NC + lax.axis_index("c")
    base = wid * CHUNK
    scr_i = scr.bitcast(jnp.int32)

    pltpu.async_copy(
        inp_t_hbm.at[:, pl.ds(base, CHUNK)], scr.at[pl.ds(0, 2), :], sem_l
    ).wait()

    b0 = jnp.float32(_B0)
    b1 = jnp.float32(_B1)
    m0 = jnp.float32(_M0)
    m1 = jnp.float32(_M1)

    flat = data_hbm.at[0, pl.ds(0, IDX_BLK)]

    g_cps = []
    for j in range(NBLK):
        blk = pl.ds(j * IDX_BLK, IDX_BLK)
        for g in range(j * GRP_PER_BLK, (j + 1) * GRP_PER_BLK):
            grp = pl.ds(g * LANES, LANES)
            pos = scr[0, grp]
            vel = scr[1, grp]
            r = ((pos + b0) * m0).astype(jnp.int32)
            c = ((vel + b1) * m1).astype(jnp.int32)
            scr_i[2, grp] = (
                ((r >> 3) << 13) + ((r & 7) << 7) + ((c >> 7) << 10) + (c & 127)
            )
        g_cps.append(
            pltpu.async_copy(flat.at[scr_i.at[2, blk]], scr.at[3, blk], sem_g)
        )
    for cp in g_cps:
        cp.wait()

    pltpu.sync_copy(scr.at[3], out_hbm.at[pl.ds(base, CHUNK)])


def kernel(inp, data):
    return _sc_lookup(inp.T, data)


# speedup vs baseline: 1.0792x; 1.0032x over previous
import functools

import jax
import jax.numpy as jnp
from jax import lax
from jax.experimental import pallas as pl
from jax.experimental.pallas import tpu as pltpu
from jax.experimental.pallas import tpu_sc as plsc

MIN_POS = -1.2
MAX_POS = 0.6
MAX_SPEED = 0.07

N = 16384
NC = 2
NS = 16
NW = NC * NS
CHUNK = N // NW
LANES = 16
IDX_BLK = 128
NBLK = CHUNK // IDX_BLK
GRP_PER_BLK = IDX_BLK // LANES

_B0 = float(-MIN_POS)
_B1 = float(MAX_SPEED)
_M0 = float(1023.999 / (MAX_POS - MIN_POS))
_M1 = float(1023.999 / (2.0 * MAX_SPEED))

_mesh = plsc.VectorSubcoreMesh(core_axis_name="c", subcore_axis_name="s")


@functools.partial(
    pl.kernel,
    mesh=_mesh,
    out_type=jax.ShapeDtypeStruct((N,), jnp.float32),
    scratch_types=[
        pltpu.VMEM((CHUNK,), jnp.float32),
        pltpu.VMEM((CHUNK,), jnp.float32),
        pltpu.VMEM((1, CHUNK), jnp.int32),
        pltpu.VMEM((1, CHUNK), jnp.float32),
        pltpu.SemaphoreType.DMA,
        pltpu.SemaphoreType.DMA,
    ],
)
def _sc_lookup(inp_t_hbm, data_hbm, out_hbm, pos_v, vel_v, idx_v, out_v,
               sem_in, sem_g):
    wid = lax.axis_index("s") * NC + lax.axis_index("c")
    base = wid * CHUNK

    cp_p = pltpu.async_copy(inp_t_hbm.at[0, pl.ds(base, CHUNK)], pos_v, sem_in)
    cp_v = pltpu.async_copy(inp_t_hbm.at[1, pl.ds(base, CHUNK)], vel_v, sem_in)
    cp_p.wait()
    cp_v.wait()

    b0 = jnp.float32(_B0)
    b1 = jnp.float32(_B1)
    m0 = jnp.float32(_M0)
    m1 = jnp.float32(_M1)

    flat = data_hbm.at[0, pl.ds(0, IDX_BLK)]

    copies = []
    for j in range(NBLK):
        for g in range(j * GRP_PER_BLK, (j + 1) * GRP_PER_BLK):
            grp = pl.ds(g * LANES, LANES)
            pos = pos_v[grp]
            vel = vel_v[grp]
            r = ((pos + b0) * m0).astype(jnp.int32)
            c = ((vel + b1) * m1).astype(jnp.int32)
            idx_v[0, grp] = (
                ((r >> 3) << 13) + ((r & 7) << 7) + ((c >> 7) << 10) + (c & 127)
            )
        copies.append(
            pltpu.async_copy(
                flat.at[idx_v.at[0, pl.ds(j * IDX_BLK, IDX_BLK)]],
                out_v.at[0, pl.ds(j * IDX_BLK, IDX_BLK)],
                sem_g,
            )
        )
    for cp in copies:
        cp.wait()

    pltpu.sync_copy(out_v.at[0], out_hbm.at[pl.ds(base, CHUNK)])


def kernel(inp, data):
    return _sc_lookup(inp.T, data)
